# baseline (speedup 1.0000x reference)
import functools

import jax
import jax.numpy as jnp
from jax import lax
from jax.experimental import pallas as pl
from jax.experimental.pallas import tpu as pltpu
from jax.experimental.pallas import tpu_sc as plsc

N_NODES = 10000
HIDDEN = 128
N_EDGES = 320000

NC = 2
NS = 16
NW = NC * NS
CHUNK = 128
CHUNKS_PER_W = 80
E_PAD = NW * CHUNK * CHUNKS_PER_W
ROWS_PER_TILE = 632
LAST_ROWS = 600
AGG_ROWS = 15 * ROWS_PER_TILE + LAST_ROWS


def _sc_segment_sum(x, src3d, dst3d, zrows):
  mesh = plsc.VectorSubcoreMesh(core_axis_name="c", subcore_axis_name="s")

  @functools.partial(
      pl.kernel,
      out_type=jax.ShapeDtypeStruct((NC, AGG_ROWS, HIDDEN), jnp.float32),
      mesh=mesh,
      scratch_types=[
          pltpu.VMEM((CHUNKS_PER_W, CHUNK), jnp.int32),
          pltpu.VMEM((CHUNKS_PER_W, CHUNK), jnp.int32),
          pltpu.VMEM((CHUNK, HIDDEN), jnp.float32),
          pltpu.VMEM_SHARED((AGG_ROWS, HIDDEN), jnp.float32),
      ],
  )
  def seg_sum(x_hbm, src_hbm, dst_hbm, z_hbm, out_hbm, src_v, dst_v, rows_v,
              agg_sh):
    c = lax.axis_index("c")
    s = lax.axis_index("s")
    wid = s * NC + c
    pltpu.sync_copy(src_hbm.at[wid], src_v)
    pltpu.sync_copy(dst_hbm.at[wid], dst_v)
    @pl.when(s < NS - 1)
    def _():
      pltpu.sync_copy(z_hbm,
                      agg_sh.at[pl.ds(s * ROWS_PER_TILE, ROWS_PER_TILE)])
    @pl.when(s == NS - 1)
    def _():
      pltpu.sync_copy(
          z_hbm.at[pl.ds(0, LAST_ROWS)],
          agg_sh.at[pl.ds((NS - 1) * ROWS_PER_TILE, LAST_ROWS)])
    plsc.subcore_barrier()

    for g in range(CHUNKS_PER_W):
      pltpu.sync_copy(x_hbm.at[src_v.at[g]], rows_v)
      pltpu.sync_copy(rows_v, agg_sh.at[dst_v.at[g]], add=True)

    plsc.subcore_barrier()

    @pl.when(s < NS - 1)
    def _():
      pltpu.sync_copy(agg_sh.at[pl.ds(s * ROWS_PER_TILE, ROWS_PER_TILE)],
                      out_hbm.at[c, pl.ds(s * ROWS_PER_TILE, ROWS_PER_TILE)])
    @pl.when(s == NS - 1)
    def _():
      pltpu.sync_copy(agg_sh.at[pl.ds((NS - 1) * ROWS_PER_TILE, LAST_ROWS)],
                      out_hbm.at[c, pl.ds((NS - 1) * ROWS_PER_TILE, LAST_ROWS)])

  return seg_sum(x, src3d, dst3d, zrows)


def _tc_layer(x, agg, w, b2d):
  def body(x_ref, a0_ref, a1_ref, w_ref, b_ref, o_ref):
    h = x_ref[...] + a0_ref[0] + a1_ref[0]
    y = jnp.dot(h, w_ref[...], preferred_element_type=jnp.float32)
    o_ref[...] = jnp.maximum(y + b_ref[...], 0.0)

  bm = 1000
  return pl.pallas_call(
      body,
      grid=(N_NODES // bm,),
      in_specs=[
          pl.BlockSpec((bm, HIDDEN), lambda i: (i, 0)),
          pl.BlockSpec((1, bm, HIDDEN), lambda i: (0, i, 0)),
          pl.BlockSpec((1, bm, HIDDEN), lambda i: (1, i, 0)),
          pl.BlockSpec((HIDDEN, HIDDEN), lambda i: (0, 0)),
          pl.BlockSpec((1, HIDDEN), lambda i: (0, 0)),
      ],
      out_specs=pl.BlockSpec((bm, HIDDEN), lambda i: (i, 0)),
      out_shape=jax.ShapeDtypeStruct((N_NODES, HIDDEN), jnp.float32),
  )(x, agg, agg, w, b2d)


def kernel(x, edge_index, batch, W0, b0, W1, b1, W2, b2):
  src = edge_index[0]
  dst = edge_index[1]
  pad = E_PAD - N_EDGES
  src3d = jnp.concatenate(
      [src, jnp.zeros((pad,), jnp.int32)]).reshape(NW, CHUNKS_PER_W, CHUNK)
  dst3d = jnp.concatenate(
      [dst, jnp.full((pad,), N_NODES, jnp.int32)]).reshape(
          NW, CHUNKS_PER_W, CHUNK)
  zrows = jnp.zeros((ROWS_PER_TILE, HIDDEN), jnp.float32)
  for w, b in ((W0, b0), (W1, b1), (W2, b2)):
    agg = _sc_segment_sum(x, src3d, dst3d, zrows)
    x = _tc_layer(x, agg, w, b.reshape(1, HIDDEN))
  return x


# speedup vs baseline: 1.0295x; 1.0295x over previous
import functools

import jax
import jax.numpy as jnp
from jax import lax
from jax.experimental import pallas as pl
from jax.experimental.pallas import tpu as pltpu
from jax.experimental.pallas import tpu_sc as plsc

N_NODES = 10000
HIDDEN = 128
N_EDGES = 320000

NC = 2
NS = 16
NW = NC * NS
CHUNK = 128
CHUNKS_PER_W = 80
E_PAD = NW * CHUNK * CHUNKS_PER_W
ROWS_PER_TILE = 632
LAST_ROWS = 600
AGG_ROWS = 15 * ROWS_PER_TILE + LAST_ROWS


def _sc_segment_sum(x, src3d, dst3d, zrows):
  mesh = plsc.VectorSubcoreMesh(core_axis_name="c", subcore_axis_name="s")

  @functools.partial(
      pl.kernel,
      out_type=jax.ShapeDtypeStruct((NC, AGG_ROWS, HIDDEN), jnp.float32),
      mesh=mesh,
      scratch_types=[
          pltpu.VMEM((CHUNKS_PER_W, CHUNK), jnp.int32),
          pltpu.VMEM((CHUNKS_PER_W, CHUNK), jnp.int32),
          pltpu.VMEM((CHUNK, HIDDEN), jnp.float32),
          pltpu.VMEM_SHARED((AGG_ROWS, HIDDEN), jnp.float32),
      ],
  )
  def seg_sum(x_hbm, src_hbm, dst_hbm, z_hbm, out_hbm, src_v, dst_v, rows_v,
              agg_sh):
    c = lax.axis_index("c")
    s = lax.axis_index("s")
    wid = s * NC + c
    pltpu.sync_copy(src_hbm.at[wid], src_v)
    pltpu.sync_copy(dst_hbm.at[wid], dst_v)
    @pl.when(s < NS - 1)
    def _():
      pltpu.sync_copy(z_hbm,
                      agg_sh.at[pl.ds(s * ROWS_PER_TILE, ROWS_PER_TILE)])
    @pl.when(s == NS - 1)
    def _():
      pltpu.sync_copy(
          z_hbm.at[pl.ds(0, LAST_ROWS)],
          agg_sh.at[pl.ds((NS - 1) * ROWS_PER_TILE, LAST_ROWS)])
    plsc.subcore_barrier()

    def body(g, carry):
      pltpu.sync_copy(x_hbm.at[src_v.at[g]], rows_v)
      pltpu.sync_copy(rows_v, agg_sh.at[dst_v.at[g]], add=True)
      return carry

    lax.fori_loop(0, CHUNKS_PER_W, body, 0)

    plsc.subcore_barrier()

    @pl.when(s < NS - 1)
    def _():
      pltpu.sync_copy(agg_sh.at[pl.ds(s * ROWS_PER_TILE, ROWS_PER_TILE)],
                      out_hbm.at[c, pl.ds(s * ROWS_PER_TILE, ROWS_PER_TILE)])
    @pl.when(s == NS - 1)
    def _():
      pltpu.sync_copy(agg_sh.at[pl.ds((NS - 1) * ROWS_PER_TILE, LAST_ROWS)],
                      out_hbm.at[c, pl.ds((NS - 1) * ROWS_PER_TILE, LAST_ROWS)])

  return seg_sum(x, src3d, dst3d, zrows)


def _tc_layer(x, agg, w, b2d):
  def body(x_ref, a0_ref, a1_ref, w_ref, b_ref, o_ref):
    h = x_ref[...] + a0_ref[0] + a1_ref[0]
    y = jnp.dot(h, w_ref[...], preferred_element_type=jnp.float32)
    o_ref[...] = jnp.maximum(y + b_ref[...], 0.0)

  bm = 1000
  return pl.pallas_call(
      body,
      grid=(N_NODES // bm,),
      in_specs=[
          pl.BlockSpec((bm, HIDDEN), lambda i: (i, 0)),
          pl.BlockSpec((1, bm, HIDDEN), lambda i: (0, i, 0)),
          pl.BlockSpec((1, bm, HIDDEN), lambda i: (1, i, 0)),
          pl.BlockSpec((HIDDEN, HIDDEN), lambda i: (0, 0)),
          pl.BlockSpec((1, HIDDEN), lambda i: (0, 0)),
      ],
      out_specs=pl.BlockSpec((bm, HIDDEN), lambda i: (i, 0)),
      out_shape=jax.ShapeDtypeStruct((N_NODES, HIDDEN), jnp.float32),
  )(x, agg, agg, w, b2d)


def kernel(x, edge_index, batch, W0, b0, W1, b1, W2, b2):
  src = edge_index[0]
  dst = edge_index[1]
  pad = E_PAD - N_EDGES
  src3d = jnp.concatenate(
      [src, jnp.zeros((pad,), jnp.int32)]).reshape(NW, CHUNKS_PER_W, CHUNK)
  dst3d = jnp.concatenate(
      [dst, jnp.full((pad,), N_NODES, jnp.int32)]).reshape(
          NW, CHUNKS_PER_W, CHUNK)
  zrows = jnp.zeros((ROWS_PER_TILE, HIDDEN), jnp.float32)
  for w, b in ((W0, b0), (W1, b1), (W2, b2)):
    agg = _sc_segment_sum(x, src3d, dst3d, zrows)
    x = _tc_layer(x, agg, w, b.reshape(1, HIDDEN))
  return x


# speedup vs baseline: 1.0356x; 1.0059x over previous
import functools

import jax
import jax.numpy as jnp
from jax import lax
from jax.experimental import pallas as pl
from jax.experimental.pallas import tpu as pltpu
from jax.experimental.pallas import tpu_sc as plsc

N_NODES = 10000
HIDDEN = 128
N_EDGES = 320000

NC = 2
NS = 16
NW = NC * NS
CHUNK = 128
CHUNKS_PER_W = 80
E_PAD = NW * CHUNK * CHUNKS_PER_W
ROWS_PER_TILE = 632
AGG_ROWS = NS * ROWS_PER_TILE


def _sc_segment_sum(x, src3d, dst3d, zrows):
  mesh = plsc.VectorSubcoreMesh(core_axis_name="c", subcore_axis_name="s")

  @functools.partial(
      pl.kernel,
      out_type=jax.ShapeDtypeStruct((NC, AGG_ROWS, HIDDEN), jnp.float32),
      mesh=mesh,
      scratch_types=[
          pltpu.VMEM((CHUNKS_PER_W, CHUNK), jnp.int32),
          pltpu.VMEM((CHUNKS_PER_W, CHUNK), jnp.int32),
          pltpu.VMEM((CHUNK, HIDDEN), jnp.float32),
          pltpu.VMEM_SHARED((AGG_ROWS, HIDDEN), jnp.float32),
      ],
  )
  def seg_sum(x_hbm, src_hbm, dst_hbm, z_hbm, out_hbm, src_v, dst_v, rows_v,
              agg_sh):
    c = lax.axis_index("c")
    s = lax.axis_index("s")
    wid = s * NC + c
    pltpu.sync_copy(src_hbm.at[wid], src_v)
    pltpu.sync_copy(dst_hbm.at[wid], dst_v)
    pltpu.sync_copy(z_hbm, agg_sh.at[pl.ds(s * ROWS_PER_TILE, ROWS_PER_TILE)])
    plsc.subcore_barrier()

    def body(g, carry):
      pltpu.sync_copy(x_hbm.at[src_v.at[g]], rows_v)
      pltpu.sync_copy(rows_v, agg_sh.at[dst_v.at[g]], add=True)
      return carry

    lax.fori_loop(0, CHUNKS_PER_W, body, 0)

    plsc.subcore_barrier()
    pltpu.sync_copy(agg_sh.at[pl.ds(s * ROWS_PER_TILE, ROWS_PER_TILE)],
                    out_hbm.at[c, pl.ds(s * ROWS_PER_TILE, ROWS_PER_TILE)])

  return seg_sum(x, src3d, dst3d, zrows)


def _tc_layer(x, agg, w, b2d):
  def body(x_ref, a0_ref, a1_ref, w_ref, b_ref, o_ref):
    h = x_ref[...] + a0_ref[0] + a1_ref[0]
    y = jnp.dot(h, w_ref[...], preferred_element_type=jnp.float32)
    o_ref[...] = jnp.maximum(y + b_ref[...], 0.0)

  bm = 1000
  return pl.pallas_call(
      body,
      grid=(N_NODES // bm,),
      in_specs=[
          pl.BlockSpec((bm, HIDDEN), lambda i: (i, 0)),
          pl.BlockSpec((1, bm, HIDDEN), lambda i: (0, i, 0)),
          pl.BlockSpec((1, bm, HIDDEN), lambda i: (1, i, 0)),
          pl.BlockSpec((HIDDEN, HIDDEN), lambda i: (0, 0)),
          pl.BlockSpec((1, HIDDEN), lambda i: (0, 0)),
      ],
      out_specs=pl.BlockSpec((bm, HIDDEN), lambda i: (i, 0)),
      out_shape=jax.ShapeDtypeStruct((N_NODES, HIDDEN), jnp.float32),
  )(x, agg, agg, w, b2d)


def kernel(x, edge_index, batch, W0, b0, W1, b1, W2, b2):
  src = edge_index[0]
  dst = edge_index[1]
  pad = E_PAD - N_EDGES
  src3d = jnp.concatenate(
      [src, jnp.zeros((pad,), jnp.int32)]).reshape(NW, CHUNKS_PER_W, CHUNK)
  dst3d = jnp.concatenate(
      [dst, jnp.full((pad,), N_NODES, jnp.int32)]).reshape(
          NW, CHUNKS_PER_W, CHUNK)
  zrows = jnp.zeros((ROWS_PER_TILE, HIDDEN), jnp.float32)
  for w, b in ((W0, b0), (W1, b1), (W2, b2)):
    agg = _sc_segment_sum(x, src3d, dst3d, zrows)
    x = _tc_layer(x, agg, w, b.reshape(1, HIDDEN))
  return x


# speedup vs baseline: 1.5852x; 1.5307x over previous
import functools

import jax
import jax.numpy as jnp
from jax import lax
from jax.experimental import pallas as pl
from jax.experimental.pallas import tpu as pltpu
from jax.experimental.pallas import tpu_sc as plsc

N_NODES = 10000
HIDDEN = 128
N_EDGES = 320000

NC = 2
NS = 16
NW = NC * NS
CHUNK = 128
CHUNKS_PER_W = 79
E_PAD = NW * CHUNK * CHUNKS_PER_W
ROWS_PER_TILE = 632
AGG_ROWS = NS * ROWS_PER_TILE


def _sc_segment_sum(x, src3d, dst3d, zrows):
  mesh = plsc.VectorSubcoreMesh(core_axis_name="c", subcore_axis_name="s")

  @functools.partial(
      pl.kernel,
      out_type=jax.ShapeDtypeStruct((NC, AGG_ROWS, HIDDEN), jnp.float32),
      mesh=mesh,
      scratch_types=[
          pltpu.VMEM((CHUNKS_PER_W, CHUNK), jnp.int32),
          pltpu.VMEM((CHUNKS_PER_W, CHUNK), jnp.int32),
          pltpu.VMEM((CHUNK, HIDDEN), jnp.float32),
          pltpu.VMEM_SHARED((AGG_ROWS, HIDDEN), jnp.float32),
      ],
  )
  def seg_sum(x_hbm, src_hbm, dst_hbm, z_hbm, out_hbm, src_v, dst_v, rows_v,
              agg_sh):
    c = lax.axis_index("c")
    s = lax.axis_index("s")
    wid = s * NC + c
    pltpu.sync_copy(src_hbm.at[wid], src_v)
    pltpu.sync_copy(dst_hbm.at[wid], dst_v)
    pltpu.sync_copy(z_hbm, agg_sh.at[pl.ds(s * ROWS_PER_TILE, ROWS_PER_TILE)])
    plsc.subcore_barrier()

    def body(g, carry):
      pltpu.sync_copy(x_hbm.at[src_v.at[g]], rows_v)
      pltpu.sync_copy(rows_v, agg_sh.at[dst_v.at[g]], add=True)
      return carry

    lax.fori_loop(0, CHUNKS_PER_W, body, 0)

    plsc.subcore_barrier()
    pltpu.sync_copy(agg_sh.at[pl.ds(s * ROWS_PER_TILE, ROWS_PER_TILE)],
                    out_hbm.at[c, pl.ds(s * ROWS_PER_TILE, ROWS_PER_TILE)])

  return seg_sum(x, src3d, dst3d, zrows)


def _tc_layer(x, agg, w, b2d):
  def body(x_ref, a0_ref, a1_ref, w_ref, b_ref, o_ref):
    h = x_ref[...] + a0_ref[0] + a1_ref[0]
    y = jnp.dot(h, w_ref[...], preferred_element_type=jnp.float32)
    o_ref[...] = jnp.maximum(y + b_ref[...], 0.0)

  bm = 1000
  return pl.pallas_call(
      body,
      grid=(N_NODES // bm,),
      in_specs=[
          pl.BlockSpec((bm, HIDDEN), lambda i: (i, 0)),
          pl.BlockSpec((1, bm, HIDDEN), lambda i: (0, i, 0)),
          pl.BlockSpec((1, bm, HIDDEN), lambda i: (1, i, 0)),
          pl.BlockSpec((HIDDEN, HIDDEN), lambda i: (0, 0)),
          pl.BlockSpec((1, HIDDEN), lambda i: (0, 0)),
      ],
      out_specs=pl.BlockSpec((bm, HIDDEN), lambda i: (i, 0)),
      out_shape=jax.ShapeDtypeStruct((N_NODES, HIDDEN), jnp.float32),
  )(x, agg, agg, w, b2d)


def kernel(x, edge_index, batch, W0, b0, W1, b1, W2, b2):
  src = edge_index[0]
  dst = edge_index[1]
  pad = E_PAD - N_EDGES
  src3d = jnp.concatenate(
      [src, jnp.zeros((pad,), jnp.int32)]).reshape(NW, CHUNKS_PER_W, CHUNK)
  dst3d = jnp.concatenate(
      [dst, jnp.full((pad,), N_NODES, jnp.int32)]).reshape(
          NW, CHUNKS_PER_W, CHUNK)
  zrows = jnp.zeros((ROWS_PER_TILE, HIDDEN), jnp.float32)
  for w, b in ((W0, b0), (W1, b1), (W2, b2)):
    agg = _sc_segment_sum(x, src3d, dst3d, zrows)
    x = _tc_layer(x, agg, w, b.reshape(1, HIDDEN))
  return x


# speedup vs baseline: 1.5873x; 1.0013x over previous
import functools

import jax
import jax.numpy as jnp
from jax import lax
from jax.experimental import pallas as pl
from jax.experimental.pallas import tpu as pltpu
from jax.experimental.pallas import tpu_sc as plsc

N_NODES = 10000
HIDDEN = 128
N_EDGES = 320000

NC = 2
NS = 16
NW = NC * NS
CHUNK = 128
CHUNKS_PER_W = 79
E_PAD = NW * CHUNK * CHUNKS_PER_W
ROWS_PER_TILE = 632
AGG_ROWS = NS * ROWS_PER_TILE


def _sc_segment_sum(x, src3d, dst3d, zrows):
  mesh = plsc.VectorSubcoreMesh(core_axis_name="c", subcore_axis_name="s")

  @functools.partial(
      pl.kernel,
      out_type=jax.ShapeDtypeStruct((NC, AGG_ROWS, HIDDEN), jnp.float32),
      mesh=mesh,
      scratch_types=[
          pltpu.VMEM((CHUNKS_PER_W, CHUNK), jnp.int32),
          pltpu.VMEM((CHUNKS_PER_W, CHUNK), jnp.int32),
          pltpu.VMEM((CHUNK, HIDDEN), jnp.float32),
          pltpu.VMEM_SHARED((AGG_ROWS, HIDDEN), jnp.float32),
      ],
  )
  def seg_sum(x_hbm, src_hbm, dst_hbm, z_hbm, out_hbm, src_v, dst_v, rows_v,
              agg_sh):
    c = lax.axis_index("c")
    s = lax.axis_index("s")
    wid = s * NC + c
    pltpu.sync_copy(src_hbm.at[wid], src_v)
    pltpu.sync_copy(dst_hbm.at[wid], dst_v)
    pltpu.sync_copy(z_hbm, agg_sh.at[pl.ds(s * ROWS_PER_TILE, ROWS_PER_TILE)])
    plsc.subcore_barrier()

    def body(g, carry):
      pltpu.sync_copy(x_hbm.at[src_v.at[g]], rows_v)
      pltpu.sync_copy(rows_v, agg_sh.at[dst_v.at[g]], add=True)
      return carry

    lax.fori_loop(0, CHUNKS_PER_W, body, 0)

    plsc.subcore_barrier()
    pltpu.sync_copy(agg_sh.at[pl.ds(s * ROWS_PER_TILE, ROWS_PER_TILE)],
                    out_hbm.at[c, pl.ds(s * ROWS_PER_TILE, ROWS_PER_TILE)])

  return seg_sum(x, src3d, dst3d, zrows)


def _tc_layer(x, agg, w, b2d):
  def body(x_ref, a0_ref, a1_ref, w_ref, b_ref, o_ref):
    h = x_ref[...] + a0_ref[0] + a1_ref[0]
    y = jnp.dot(h, w_ref[...], preferred_element_type=jnp.float32)
    o_ref[...] = jnp.maximum(y + b_ref[...], 0.0)

  bm = 1000
  return pl.pallas_call(
      body,
      grid=(N_NODES // bm,),
      in_specs=[
          pl.BlockSpec((bm, HIDDEN), lambda i: (i, 0)),
          pl.BlockSpec((1, bm, HIDDEN), lambda i: (0, i, 0)),
          pl.BlockSpec((1, bm, HIDDEN), lambda i: (1, i, 0)),
          pl.BlockSpec((HIDDEN, HIDDEN), lambda i: (0, 0)),
          pl.BlockSpec((1, HIDDEN), lambda i: (0, 0)),
      ],
      out_specs=pl.BlockSpec((bm, HIDDEN), lambda i: (i, 0)),
      out_shape=jax.ShapeDtypeStruct((N_NODES, HIDDEN), jnp.float32),
  )(x, agg, agg, w, b2d)


def kernel(x, edge_index, batch, W0, b0, W1, b1, W2, b2):
  src = edge_index[0]
  dst = edge_index[1]
  pad = E_PAD - N_EDGES
  src3d = jnp.concatenate(
      [src, jnp.zeros((pad,), jnp.int32)]).reshape(NW, CHUNKS_PER_W, CHUNK)
  pad_dst = N_NODES + (jnp.arange(pad, dtype=jnp.int32) % (AGG_ROWS - N_NODES))
  dst3d = jnp.concatenate([dst, pad_dst]).reshape(NW, CHUNKS_PER_W, CHUNK)
  zrows = jnp.zeros((ROWS_PER_TILE, HIDDEN), jnp.float32)
  for w, b in ((W0, b0), (W1, b1), (W2, b2)):
    agg = _sc_segment_sum(x, src3d, dst3d, zrows)
    x = _tc_layer(x, agg, w, b.reshape(1, HIDDEN))
  return x
